# baseline (speedup 1.0000x reference)
import functools

import jax
import jax.numpy as jnp
from jax import lax
from jax.experimental import pallas as pl
from jax.experimental.pallas import tpu as pltpu
from jax.experimental.pallas import tpu_sc as plsc

B = 4096
M = 50
EMB = 64

NC = 2
NS = 16
NW = NC * NS

BPW = B // NW
RPW = (B * M) // NW
CHUNK = 640
NCHUNK = RPW // CHUNK


def _sc_gather(user_id, item_id_p, item_id_n, inter_id_flat, user_emb, item_emb):
    mesh = plsc.VectorSubcoreMesh(
        core_axis_name="c", subcore_axis_name="s", num_cores=NC, num_subcores=NS
    )

    @functools.partial(
        pl.kernel,
        out_type=(
            jax.ShapeDtypeStruct((B, EMB), jnp.float32),
            jax.ShapeDtypeStruct((B, EMB), jnp.float32),
            jax.ShapeDtypeStruct((B, EMB), jnp.float32),
            jax.ShapeDtypeStruct((B * M, EMB), jnp.float32),
        ),
        mesh=mesh,
        scratch_types=[
            pltpu.VMEM((BPW,), jnp.int32),
            pltpu.VMEM((BPW, EMB), jnp.float32),
            pltpu.VMEM((CHUNK,), jnp.int32),
            pltpu.VMEM((CHUNK, EMB), jnp.float32),
            pltpu.SemaphoreType.DMA,
        ],
        compiler_params=pltpu.CompilerParams(use_tc_tiling_on_sc=False),
    )
    def gather_kernel(uid_hbm, ipid_hbm, inid_hbm, iid_hbm, uemb_hbm, iemb_hbm,
                      u_out, ip_out, in_out, inter_out,
                      idx_s, rows_s, idx_c, rows_c, sem):
        wid = lax.axis_index("s") * NC + lax.axis_index("c")
        base = wid * BPW
        for ids_hbm, table_hbm, out_hbm in (
            (uid_hbm, uemb_hbm, u_out),
            (ipid_hbm, iemb_hbm, ip_out),
            (inid_hbm, iemb_hbm, in_out),
        ):
            pltpu.sync_copy(ids_hbm.at[pl.ds(base, BPW)], idx_s)
            pltpu.async_copy(table_hbm.at[idx_s], rows_s, sem).wait()
            pltpu.sync_copy(rows_s, out_hbm.at[pl.ds(base, BPW)])
        rbase = wid * RPW
        for k in range(NCHUNK):
            cbase = rbase + k * CHUNK
            pltpu.sync_copy(iid_hbm.at[pl.ds(cbase, CHUNK)], idx_c)
            pltpu.async_copy(iemb_hbm.at[idx_c], rows_c, sem).wait()
            pltpu.sync_copy(rows_c, inter_out.at[pl.ds(cbase, CHUNK)])

    return gather_kernel(user_id, item_id_p, item_id_n, inter_id_flat,
                         user_emb, item_emb)


def _renorm(v):
    n = jnp.sqrt(jnp.sum(v * v, axis=-1, keepdims=True))
    return v * jnp.minimum(1.0, 1.0 / jnp.maximum(n, 1e-7))


def _attn_body(u_ref, ip_ref, in_ref, inter_ref, kw_ref, vw_ref,
               uf_ref, ipf_ref, inf_ref, relp_ref, reln_ref):
    ws = jnp.sum(kw_ref[...] * vw_ref[...], axis=0)
    u = _renorm(u_ref[...])
    uf_ref[...] = u
    inter = inter_ref[...]
    n2 = jnp.sum(inter * inter, axis=-1, keepdims=True)
    interf = inter * jnp.minimum(1.0, 1.0 / jnp.maximum(jnp.sqrt(n2), 1e-7))
    for it_ref, itf_ref, rel_ref in ((ip_ref, ipf_ref, relp_ref),
                                     (in_ref, inf_ref, reln_ref)):
        it = _renorm(it_ref[...])
        itf_ref[...] = it
        t = it * ws[None, :]
        s = jnp.sum(interf * (t * u)[:, None, :], axis=-1)
        smax = jnp.max(s, axis=1, keepdims=True)
        e = jnp.exp(s - smax)
        attn = e / jnp.sum(e, axis=1, keepdims=True)
        rel_ref[...] = jnp.sum(interf * attn[:, :, None], axis=1) * t


def _tc_attention(raw_u, raw_ip, raw_in, raw_inter3d, key_w, val_w):
    Bb = 256
    grid = (B // Bb,)
    row_spec = pl.BlockSpec((Bb, EMB), lambda i: (i, 0))
    w_spec = pl.BlockSpec((EMB, EMB), lambda i: (0, 0))
    return pl.pallas_call(
        _attn_body,
        grid=grid,
        in_specs=[
            row_spec, row_spec, row_spec,
            pl.BlockSpec((Bb, M, EMB), lambda i: (i, 0, 0)),
            w_spec, w_spec,
        ],
        out_specs=[row_spec, row_spec, row_spec, row_spec, row_spec],
        out_shape=[jax.ShapeDtypeStruct((B, EMB), jnp.float32)] * 5,
    )(raw_u, raw_ip, raw_in, raw_inter3d, key_w, val_w)


def kernel(user_id, item_id_p, item_id_n, inter_id, user_emb, item_emb, key_w, val_w):
    uid = user_id.astype(jnp.int32)
    ipid = item_id_p.astype(jnp.int32)
    inid = item_id_n.astype(jnp.int32)
    iid = inter_id.reshape(-1).astype(jnp.int32)
    raw_u, raw_ip, raw_in, raw_inter = _sc_gather(
        uid, ipid, inid, iid, user_emb, item_emb)
    uf, ipf, inf, relp, reln = _tc_attention(
        raw_u, raw_ip, raw_in, raw_inter.reshape(B, M, EMB), key_w, val_w)
    return (uf, ipf, inf, relp, reln)


# speedup vs baseline: 1.0411x; 1.0411x over previous
import functools

import jax
import jax.numpy as jnp
from jax import lax
from jax.experimental import pallas as pl
from jax.experimental.pallas import tpu as pltpu
from jax.experimental.pallas import tpu_sc as plsc

B = 4096
M = 50
EMB = 64

NC = 2
NS = 16
NW = NC * NS

BPW = B // NW
RPW = (B * M) // NW
CHUNK = 640
NCHUNK = RPW // CHUNK


def _sc_gather(user_id, item_id_p, item_id_n, inter_id_flat, user_emb, item_emb):
    mesh = plsc.VectorSubcoreMesh(
        core_axis_name="c", subcore_axis_name="s", num_cores=NC, num_subcores=NS
    )

    @functools.partial(
        pl.kernel,
        out_type=(
            jax.ShapeDtypeStruct((B, EMB), jnp.float32),
            jax.ShapeDtypeStruct((B, EMB), jnp.float32),
            jax.ShapeDtypeStruct((B, EMB), jnp.float32),
            jax.ShapeDtypeStruct((B * M, EMB), jnp.float32),
        ),
        mesh=mesh,
        scratch_types=[
            pltpu.VMEM((BPW,), jnp.int32),
            pltpu.VMEM((BPW, EMB), jnp.float32),
            pltpu.VMEM((CHUNK,), jnp.int32),
            pltpu.VMEM((CHUNK, EMB), jnp.float32),
            pltpu.SemaphoreType.DMA,
        ],
        compiler_params=pltpu.CompilerParams(use_tc_tiling_on_sc=False),
    )
    def gather_kernel(uid_hbm, ipid_hbm, inid_hbm, iid_hbm, uemb_hbm, iemb_hbm,
                      u_out, ip_out, in_out, inter_out,
                      idx_s, rows_s, idx_c, rows_c, sem):
        wid = lax.axis_index("s") * NC + lax.axis_index("c")
        base = wid * BPW
        for ids_hbm, table_hbm, out_hbm in (
            (uid_hbm, uemb_hbm, u_out),
            (ipid_hbm, iemb_hbm, ip_out),
            (inid_hbm, iemb_hbm, in_out),
        ):
            pltpu.sync_copy(ids_hbm.at[pl.ds(base, BPW)], idx_s)
            pltpu.async_copy(table_hbm.at[idx_s], rows_s, sem).wait()
            pltpu.sync_copy(rows_s, out_hbm.at[pl.ds(base, BPW)])
        rbase = wid * RPW
        for k in range(NCHUNK):
            cbase = rbase + k * CHUNK
            pltpu.sync_copy(iid_hbm.at[pl.ds(cbase, CHUNK)], idx_c)
            pltpu.async_copy(iemb_hbm.at[idx_c], rows_c, sem).wait()
            pltpu.sync_copy(rows_c, inter_out.at[pl.ds(cbase, CHUNK)])

    return gather_kernel(user_id, item_id_p, item_id_n, inter_id_flat,
                         user_emb, item_emb)


def _renorm(v):
    n = jnp.sqrt(jnp.sum(v * v, axis=-1, keepdims=True))
    return v * jnp.minimum(1.0, 1.0 / jnp.maximum(n, 1e-7))


def _attn_body(u_ref, ip_ref, in_ref, inter_ref, kw_ref, vw_ref,
               uf_ref, ipf_ref, inf_ref, relp_ref, reln_ref):
    ws = jnp.sum(kw_ref[...] * vw_ref[...], axis=0)
    u = _renorm(u_ref[...])
    uf_ref[...] = u
    x = inter_ref[...]
    n2 = jnp.sum(x * x, axis=-1, keepdims=True)
    scale3 = jnp.minimum(1.0, 1.0 / jnp.maximum(jnp.sqrt(n2), 1e-7))
    for it_ref, itf_ref, rel_ref in ((ip_ref, ipf_ref, relp_ref),
                                     (in_ref, inf_ref, reln_ref)):
        it = _renorm(it_ref[...])
        itf_ref[...] = it
        t = it * ws[None, :]
        s = jnp.sum(x * (t * u)[:, None, :], axis=-1, keepdims=True) * scale3
        smax = jnp.max(s, axis=1, keepdims=True)
        e = jnp.exp(s - smax)
        attn = e * scale3 / jnp.sum(e, axis=1, keepdims=True)
        rel_ref[...] = jnp.sum(x * attn, axis=1) * t


def _tc_attention(raw_u, raw_ip, raw_in, raw_inter3d, key_w, val_w):
    Bb = 256
    grid = (B // Bb,)
    row_spec = pl.BlockSpec((Bb, EMB), lambda i: (i, 0))
    w_spec = pl.BlockSpec((EMB, EMB), lambda i: (0, 0))
    return pl.pallas_call(
        _attn_body,
        grid=grid,
        in_specs=[
            row_spec, row_spec, row_spec,
            pl.BlockSpec((Bb, M, EMB), lambda i: (i, 0, 0)),
            w_spec, w_spec,
        ],
        out_specs=[row_spec, row_spec, row_spec, row_spec, row_spec],
        out_shape=[jax.ShapeDtypeStruct((B, EMB), jnp.float32)] * 5,
    )(raw_u, raw_ip, raw_in, raw_inter3d, key_w, val_w)


def kernel(user_id, item_id_p, item_id_n, inter_id, user_emb, item_emb, key_w, val_w):
    uid = user_id.astype(jnp.int32)
    ipid = item_id_p.astype(jnp.int32)
    inid = item_id_n.astype(jnp.int32)
    iid = inter_id.reshape(-1).astype(jnp.int32)
    raw_u, raw_ip, raw_in, raw_inter = _sc_gather(
        uid, ipid, inid, iid, user_emb, item_emb)
    uf, ipf, inf, relp, reln = _tc_attention(
        raw_u, raw_ip, raw_in, raw_inter.reshape(B, M, EMB), key_w, val_w)
    return (uf, ipf, inf, relp, reln)


# speedup vs baseline: 1.3078x; 1.2562x over previous
import functools

import jax
import jax.numpy as jnp
from jax import lax
from jax.experimental import pallas as pl
from jax.experimental.pallas import tpu as pltpu
from jax.experimental.pallas import tpu_sc as plsc

B = 4096
M = 50
MP = 56
EMB = 64
LANES = 128

NC = 2
NS = 16
NW = NC * NS

BPW = B // NW
CB = 8
NCH = (B // NW) // CB


def _sc_gather(ipid, inid, iid_flat, item_emb_lin):
    mesh = plsc.VectorSubcoreMesh(
        core_axis_name="c", subcore_axis_name="s", num_cores=NC, num_subcores=NS
    )

    @functools.partial(
        pl.kernel,
        out_type=(
            jax.ShapeDtypeStruct((B, EMB), jnp.float32),
            jax.ShapeDtypeStruct((B, EMB), jnp.float32),
            jax.ShapeDtypeStruct((B, MP, LANES), jnp.float32),
        ),
        mesh=mesh,
        scratch_types=[
            pltpu.VMEM((BPW,), jnp.int32),
            pltpu.VMEM((BPW, EMB), jnp.float32),
            pltpu.VMEM((CB * M,), jnp.int32),
            pltpu.VMEM((CB * M, EMB), jnp.float32),
            pltpu.SemaphoreType.DMA,
            pltpu.SemaphoreType.DMA,
        ],
        compiler_params=pltpu.CompilerParams(use_tc_tiling_on_sc=False),
    )
    def gather(ipid_h, inid_h, iid_h, iemb_h, ip_out, in_out, x_out,
               idx_s, rows_s, idx_c, rows_c, gsem, wsem):
        wid = lax.axis_index("s") * NC + lax.axis_index("c")
        base = wid * BPW
        for ids_h, out_h in ((ipid_h, ip_out), (inid_h, in_out)):
            pltpu.sync_copy(ids_h.at[pl.ds(base, BPW)], idx_s)
            pltpu.async_copy(iemb_h.at[idx_s], rows_s, gsem).wait()
            pltpu.sync_copy(rows_s, out_h.at[pl.ds(base, BPW)])
        for c in range(NCH):
            b0 = base + c * CB
            pltpu.sync_copy(iid_h.at[pl.ds(b0 * M, CB * M)], idx_c)
            pltpu.async_copy(iemb_h.at[idx_c], rows_c, gsem).wait()
            cps = [
                pltpu.async_copy(
                    rows_c.at[pl.ds(j * M, M)],
                    x_out.at[b0 + j, pl.ds(0, M), pl.ds(0, EMB)],
                    wsem,
                )
                for j in range(CB)
            ]
            for cp in cps:
                cp.wait()

    return gather(ipid, inid, iid_flat, item_emb_lin)


def _renorm(v):
    n2 = jnp.sum(v * v, axis=-1, keepdims=True)
    return v * jnp.minimum(1.0, jax.lax.rsqrt(jnp.maximum(n2, 1e-14)))


def _attn_body(uid_ref, ip_ref, in_ref, x_ref, kw_ref, vw_ref, uemb_ref,
               uf_ref, ipf_ref, inf_ref, relp_ref, reln_ref, ubuf, sem):
    Bb = ubuf.shape[0]

    def issue(i, _):
        pltpu.make_async_copy(
            uemb_ref.at[pl.ds(uid_ref[i, 0], 1), :],
            ubuf.at[pl.ds(i, 1), :], sem).start()
        return 0

    lax.fori_loop(0, Bb, issue, 0)

    ws = jnp.sum(kw_ref[...] * vw_ref[...], axis=0)
    xr = x_ref[...]
    lane_ok = jax.lax.broadcasted_iota(jnp.int32, (1, 1, LANES), 2) < EMB
    row_ok3 = jax.lax.broadcasted_iota(jnp.int32, (1, MP, 1), 1) < M
    x = jnp.where(jnp.logical_and(lane_ok, row_ok3), xr, 0.0)
    n2 = jnp.sum(x * x, axis=-1, keepdims=True)
    scale3 = jnp.minimum(1.0, jax.lax.rsqrt(jnp.maximum(n2, 1e-14)))

    pltpu.make_async_copy(uemb_ref.at[pl.ds(0, Bb), :], ubuf, sem).wait()
    u = _renorm(ubuf[...])
    uf_ref[...] = u

    zpad = jnp.zeros_like(u)
    for it_ref, itf_ref, rel_ref in ((ip_ref, ipf_ref, relp_ref),
                                     (in_ref, inf_ref, reln_ref)):
        it = _renorm(it_ref[...])
        itf_ref[...] = it
        t = it * ws[None, :]
        c2 = jnp.concatenate([t * u, zpad], axis=-1)
        s = jnp.sum(x * c2[:, None, :], axis=-1, keepdims=True) * scale3
        e = jnp.where(row_ok3, jnp.exp(s), 0.0)
        r = jax.lax.reciprocal(jnp.sum(e, axis=1, keepdims=True))
        attn = e * (scale3 * r)
        rel_ref[...] = jnp.sum(x * attn, axis=1)[:, :EMB] * t


def _tc_attention(uid2, raw_ip, raw_in, xpad, key_w, val_w, user_emb):
    Bb = 256
    grid = (B // Bb,)
    row_spec = pl.BlockSpec((Bb, EMB), lambda i: (i, 0))
    w_spec = pl.BlockSpec((EMB, EMB), lambda i: (0, 0))
    return pl.pallas_call(
        _attn_body,
        grid=grid,
        in_specs=[
            pl.BlockSpec((Bb, 1), lambda i: (i, 0), memory_space=pltpu.SMEM),
            row_spec, row_spec,
            pl.BlockSpec((Bb, MP, LANES), lambda i: (i, 0, 0)),
            w_spec, w_spec,
            pl.BlockSpec(memory_space=pl.ANY),
        ],
        out_specs=[row_spec, row_spec, row_spec, row_spec, row_spec],
        out_shape=[jax.ShapeDtypeStruct((B, EMB), jnp.float32)] * 5,
        scratch_shapes=[
            pltpu.VMEM((Bb, EMB), jnp.float32),
            pltpu.SemaphoreType.DMA,
        ],
    )(uid2, raw_ip, raw_in, xpad, key_w, val_w, user_emb)


def kernel(user_id, item_id_p, item_id_n, inter_id, user_emb, item_emb, key_w, val_w):
    uid = user_id.astype(jnp.int32)
    ipid = item_id_p.astype(jnp.int32)
    inid = item_id_n.astype(jnp.int32)
    iid = inter_id.reshape(-1).astype(jnp.int32)
    raw_ip, raw_in, xpad = _sc_gather(ipid, inid, iid, item_emb)
    uf, ipf, inf, relp, reln = _tc_attention(
        uid.reshape(B, 1), raw_ip, raw_in, xpad, key_w, val_w, user_emb)
    return (uf, ipf, inf, relp, reln)


# speedup vs baseline: 1.3149x; 1.0054x over previous
import functools

import jax
import jax.numpy as jnp
from jax import lax
from jax.experimental import pallas as pl
from jax.experimental.pallas import tpu as pltpu
from jax.experimental.pallas import tpu_sc as plsc

B = 4096
M = 50
MP = 56
EMB = 64
LANES = 128

NC = 2
NS = 16
NW = NC * NS

BPW = B // NW
CB = 8
NCH = (B // NW) // CB


def _sc_gather(ipid, inid, iid_flat, item_emb_lin):
    mesh = plsc.VectorSubcoreMesh(
        core_axis_name="c", subcore_axis_name="s", num_cores=NC, num_subcores=NS
    )

    @functools.partial(
        pl.kernel,
        out_type=(
            jax.ShapeDtypeStruct((B, EMB), jnp.float32),
            jax.ShapeDtypeStruct((B, EMB), jnp.float32),
            jax.ShapeDtypeStruct((B, MP, LANES), jnp.float32),
        ),
        mesh=mesh,
        scratch_types=[
            pltpu.VMEM((BPW,), jnp.int32),
            pltpu.VMEM((BPW, EMB), jnp.float32),
            pltpu.VMEM((CB * M,), jnp.int32),
            pltpu.VMEM((CB * M, EMB), jnp.float32),
            pltpu.SemaphoreType.DMA,
            pltpu.SemaphoreType.DMA,
        ],
        compiler_params=pltpu.CompilerParams(use_tc_tiling_on_sc=False),
    )
    def gather(ipid_h, inid_h, iid_h, iemb_h, ip_out, in_out, x_out,
               idx_s, rows_s, idx_c, rows_c, gsem, wsem):
        wid = lax.axis_index("s") * NC + lax.axis_index("c")
        base = wid * BPW
        for ids_h, out_h in ((ipid_h, ip_out), (inid_h, in_out)):
            pltpu.sync_copy(ids_h.at[pl.ds(base, BPW)], idx_s)
            pltpu.async_copy(iemb_h.at[idx_s], rows_s, gsem).wait()
            pltpu.sync_copy(rows_s, out_h.at[pl.ds(base, BPW)])
        for c in range(NCH):
            b0 = base + c * CB
            pltpu.sync_copy(iid_h.at[pl.ds(b0 * M, CB * M)], idx_c)
            pltpu.async_copy(iemb_h.at[idx_c], rows_c, gsem).wait()
            cps = [
                pltpu.async_copy(
                    rows_c.at[pl.ds(j * M, M)],
                    x_out.at[b0 + j, pl.ds(0, M), pl.ds(0, EMB)],
                    wsem,
                )
                for j in range(CB)
            ]
            for cp in cps:
                cp.wait()

    return gather(ipid, inid, iid_flat, item_emb_lin)


def _renorm(v):
    n2 = jnp.sum(v * v, axis=-1, keepdims=True)
    return v * jnp.minimum(1.0, jax.lax.rsqrt(jnp.maximum(n2, 1e-14)))


NSEM = 8


def _attn_body(uid_ref, ip_ref, in_ref, x_ref, kw_ref, vw_ref, uemb_ref,
               uf_ref, ipf_ref, inf_ref, relp_ref, reln_ref, ubuf, sems):
    Bb = ubuf.shape[0]

    def issue(i, _):
        for k in range(NSEM):
            r = i * NSEM + k
            pltpu.make_async_copy(
                uemb_ref.at[pl.ds(uid_ref[r, 0], 1), :],
                ubuf.at[pl.ds(r, 1), :], sems.at[k]).start()
        return 0

    lax.fori_loop(0, Bb // NSEM, issue, 0)

    ws = jnp.sum(kw_ref[...] * vw_ref[...], axis=0)
    xr = x_ref[...]
    lane_ok = jax.lax.broadcasted_iota(jnp.int32, (1, 1, LANES), 2) < EMB
    row_ok3 = jax.lax.broadcasted_iota(jnp.int32, (1, MP, 1), 1) < M
    x = jnp.where(jnp.logical_and(lane_ok, row_ok3), xr, 0.0)
    n2 = jnp.sum(x * x, axis=-1, keepdims=True)
    scale3 = jnp.minimum(1.0, jax.lax.rsqrt(jnp.maximum(n2, 1e-14)))

    for k in range(NSEM):
        pltpu.make_async_copy(
            uemb_ref.at[pl.ds(0, Bb // NSEM), :],
            ubuf.at[pl.ds(0, Bb // NSEM), :], sems.at[k]).wait()
    u = _renorm(ubuf[...])
    uf_ref[...] = u

    zpad = jnp.zeros_like(u)
    for it_ref, itf_ref, rel_ref in ((ip_ref, ipf_ref, relp_ref),
                                     (in_ref, inf_ref, reln_ref)):
        it = _renorm(it_ref[...])
        itf_ref[...] = it
        t = it * ws[None, :]
        c2 = jnp.concatenate([t * u, zpad], axis=-1)
        s = jnp.sum(x * c2[:, None, :], axis=-1, keepdims=True) * scale3
        e = jnp.where(row_ok3, jnp.exp(s), 0.0)
        r = jax.lax.reciprocal(jnp.sum(e, axis=1, keepdims=True))
        attn = e * (scale3 * r)
        rel_ref[...] = jnp.sum(x * attn, axis=1)[:, :EMB] * t


def _tc_attention(uid2, raw_ip, raw_in, xpad, key_w, val_w, user_emb):
    Bb = 256
    grid = (B // Bb,)
    row_spec = pl.BlockSpec((Bb, EMB), lambda i: (i, 0))
    w_spec = pl.BlockSpec((EMB, EMB), lambda i: (0, 0))
    return pl.pallas_call(
        _attn_body,
        grid=grid,
        in_specs=[
            pl.BlockSpec((Bb, 1), lambda i: (i, 0), memory_space=pltpu.SMEM),
            row_spec, row_spec,
            pl.BlockSpec((Bb, MP, LANES), lambda i: (i, 0, 0)),
            w_spec, w_spec,
            pl.BlockSpec(memory_space=pl.ANY),
        ],
        out_specs=[row_spec, row_spec, row_spec, row_spec, row_spec],
        out_shape=[jax.ShapeDtypeStruct((B, EMB), jnp.float32)] * 5,
        scratch_shapes=[
            pltpu.VMEM((Bb, EMB), jnp.float32),
            pltpu.SemaphoreType.DMA((NSEM,)),
        ],
    )(uid2, raw_ip, raw_in, xpad, key_w, val_w, user_emb)


def kernel(user_id, item_id_p, item_id_n, inter_id, user_emb, item_emb, key_w, val_w):
    uid = user_id.astype(jnp.int32)
    ipid = item_id_p.astype(jnp.int32)
    inid = item_id_n.astype(jnp.int32)
    iid = inter_id.reshape(-1).astype(jnp.int32)
    raw_ip, raw_in, xpad = _sc_gather(ipid, inid, iid, item_emb)
    uf, ipf, inf, relp, reln = _tc_attention(
        uid.reshape(B, 1), raw_ip, raw_in, xpad, key_w, val_w, user_emb)
    return (uf, ipf, inf, relp, reln)
